# baseline (speedup 1.0000x reference)
import jax
import jax.numpy as jnp
from jax import lax
from jax.experimental import pallas as pl
from jax.experimental.pallas import tpu as pltpu
from jax.experimental.pallas import tpu_sc as plsc

VOCAB_SIZE = 100000
EMBED_DIM = 100
BATCH = 16384

NUM_CORES = 2
NUM_SUBCORES = 16
NUM_WORKERS = NUM_CORES * NUM_SUBCORES
B_PER_W = BATCH // NUM_WORKERS


def _sc_gather_body(center_hbm, context_hbm, u_hbm, v_hbm,
                    ue_hbm, ve_hbm,
                    cidx_v, xidx_v, urows_v, vrows_v, sem_u, sem_v):
    wid = lax.axis_index("s") * NUM_CORES + lax.axis_index("c")
    base = wid * B_PER_W

    pltpu.sync_copy(center_hbm.at[pl.ds(base, B_PER_W)], cidx_v)
    pltpu.sync_copy(context_hbm.at[pl.ds(base, B_PER_W)], xidx_v)

    cp_u = pltpu.async_copy(u_hbm.at[cidx_v], urows_v, sem_u)
    cp_v = pltpu.async_copy(v_hbm.at[xidx_v], vrows_v, sem_v)
    cp_u.wait()
    cp_v.wait()

    pltpu.sync_copy(urows_v, ue_hbm.at[pl.ds(base, B_PER_W)])
    pltpu.sync_copy(vrows_v, ve_hbm.at[pl.ds(base, B_PER_W)])


def _tc_dot_body(ue_ref, ve_ref, out_ref):
    prod = ue_ref[...] * ve_ref[...]
    dots = jnp.sum(prod, axis=-1)
    out_ref[...] = 1.0 / (1.0 + jnp.exp(-dots))


TC_BLOCK = 2048


@jax.jit
def _run(center, context, u, v):
    mesh = plsc.VectorSubcoreMesh(core_axis_name="c", subcore_axis_name="s")
    ue, ve = pl.kernel(
        _sc_gather_body,
        out_type=(
            jax.ShapeDtypeStruct((BATCH, EMBED_DIM), jnp.float32),
            jax.ShapeDtypeStruct((BATCH, EMBED_DIM), jnp.float32),
        ),
        mesh=mesh,
        compiler_params=pltpu.CompilerParams(use_tc_tiling_on_sc=False),
        scratch_types=[
            pltpu.VMEM((B_PER_W,), jnp.int32),
            pltpu.VMEM((B_PER_W,), jnp.int32),
            pltpu.VMEM((B_PER_W, EMBED_DIM), jnp.float32),
            pltpu.VMEM((B_PER_W, EMBED_DIM), jnp.float32),
            pltpu.SemaphoreType.DMA,
            pltpu.SemaphoreType.DMA,
        ],
    )(center, context, u, v)

    out = pl.pallas_call(
        _tc_dot_body,
        out_shape=jax.ShapeDtypeStruct((BATCH,), jnp.float32),
        grid=(BATCH // TC_BLOCK,),
        in_specs=[
            pl.BlockSpec((TC_BLOCK, EMBED_DIM), lambda i: (i, 0)),
            pl.BlockSpec((TC_BLOCK, EMBED_DIM), lambda i: (i, 0)),
        ],
        out_specs=pl.BlockSpec((TC_BLOCK,), lambda i: (i,)),
    )(ue, ve)
    return out


def kernel(center, context, U, V):
    return _run(center, context, U, V)


# speedup vs baseline: 1.1985x; 1.1985x over previous
import jax
import jax.numpy as jnp
from jax import lax
from jax.experimental import pallas as pl
from jax.experimental.pallas import tpu as pltpu
from jax.experimental.pallas import tpu_sc as plsc

VOCAB_SIZE = 100000
EMBED_DIM = 100
PAD_DIM = 128
BATCH = 16384

NUM_CORES = 2
NUM_SUBCORES = 16
NUM_WORKERS = NUM_CORES * NUM_SUBCORES
B_PER_W = BATCH // NUM_WORKERS


CHUNK = 256
N_CHUNKS = B_PER_W // CHUNK


def _sc_gather_body(center_hbm, context_hbm, u_hbm, v_hbm,
                    ue_hbm, ve_hbm,
                    cidx_v, xidx_v, urows_v, vrows_v, sem_u, sem_v):
    wid = lax.axis_index("s") * NUM_CORES + lax.axis_index("c")
    base = wid * B_PER_W

    pltpu.sync_copy(center_hbm.at[pl.ds(base, B_PER_W)], cidx_v)
    pltpu.sync_copy(context_hbm.at[pl.ds(base, B_PER_W)], xidx_v)

    for ci in range(N_CHUNKS):
        off = ci * CHUNK
        cp_u = pltpu.async_copy(
            u_hbm.at[cidx_v.at[pl.ds(off, CHUNK)]], urows_v, sem_u)
        cp_v = pltpu.async_copy(
            v_hbm.at[xidx_v.at[pl.ds(off, CHUNK)]], vrows_v, sem_v)
        cp_u.wait()
        cp_v.wait()
        pltpu.sync_copy(urows_v, ue_hbm.at[pl.ds(base + off, CHUNK)])
        pltpu.sync_copy(vrows_v, ve_hbm.at[pl.ds(base + off, CHUNK)])


def _tc_dot_body(ue_ref, ve_ref, out_ref):
    prod = ue_ref[...] * ve_ref[...]
    dots = jnp.sum(prod, axis=-1)
    out_ref[...] = 1.0 / (1.0 + jnp.exp(-dots))


TC_BLOCK = 2048


@jax.jit
def _run(center, context, u, v):
    u_pad = jnp.pad(u, ((0, 0), (0, PAD_DIM - EMBED_DIM)))
    v_pad = jnp.pad(v, ((0, 0), (0, PAD_DIM - EMBED_DIM)))

    mesh = plsc.VectorSubcoreMesh(core_axis_name="c", subcore_axis_name="s")
    ue, ve = pl.kernel(
        _sc_gather_body,
        out_type=(
            jax.ShapeDtypeStruct((BATCH, PAD_DIM), jnp.float32),
            jax.ShapeDtypeStruct((BATCH, PAD_DIM), jnp.float32),
        ),
        mesh=mesh,
        scratch_types=[
            pltpu.VMEM((B_PER_W,), jnp.int32),
            pltpu.VMEM((B_PER_W,), jnp.int32),
            pltpu.VMEM((CHUNK, PAD_DIM), jnp.float32),
            pltpu.VMEM((CHUNK, PAD_DIM), jnp.float32),
            pltpu.SemaphoreType.DMA,
            pltpu.SemaphoreType.DMA,
        ],
    )(center, context, u_pad, v_pad)

    out = pl.pallas_call(
        _tc_dot_body,
        out_shape=jax.ShapeDtypeStruct((BATCH,), jnp.float32),
        grid=(BATCH // TC_BLOCK,),
        in_specs=[
            pl.BlockSpec((TC_BLOCK, PAD_DIM), lambda i: (i, 0)),
            pl.BlockSpec((TC_BLOCK, PAD_DIM), lambda i: (i, 0)),
        ],
        out_specs=pl.BlockSpec((TC_BLOCK,), lambda i: (i,)),
    )(ue, ve)
    return out


def kernel(center, context, U, V):
    return _run(center, context, U, V)


# speedup vs baseline: 2.5186x; 2.1015x over previous
import jax
import jax.numpy as jnp
from jax import lax
from jax.experimental import pallas as pl
from jax.experimental.pallas import tpu as pltpu
from jax.experimental.pallas import tpu_sc as plsc

VOCAB_SIZE = 100000
EMBED_DIM = 100
PAD_DIM = 128
BATCH = 16384

NUM_CORES = 2
NUM_SUBCORES = 16
NUM_WORKERS = NUM_CORES * NUM_SUBCORES
B_PER_W = BATCH // NUM_WORKERS


CHUNK = 256
N_CHUNKS = B_PER_W // CHUNK


def _sc_gather_body(center_hbm, context_hbm, u_hbm, v_hbm,
                    ue_hbm, ve_hbm,
                    cidx_v, xidx_v, urows_v, vrows_v, sem_u, sem_v):
    wid = lax.axis_index("s") * NUM_CORES + lax.axis_index("c")
    base = wid * B_PER_W

    pltpu.sync_copy(center_hbm.at[pl.ds(base, B_PER_W)], cidx_v)
    pltpu.sync_copy(context_hbm.at[pl.ds(base, B_PER_W)], xidx_v)

    for ci in range(N_CHUNKS):
        off = ci * CHUNK
        cp_u = pltpu.async_copy(
            u_hbm.at[cidx_v.at[pl.ds(off, CHUNK)]], urows_v, sem_u)
        cp_v = pltpu.async_copy(
            v_hbm.at[xidx_v.at[pl.ds(off, CHUNK)]], vrows_v, sem_v)
        cp_u.wait()
        cp_v.wait()
        pltpu.sync_copy(urows_v, ue_hbm.at[pl.ds(base + off, CHUNK)])
        pltpu.sync_copy(vrows_v, ve_hbm.at[pl.ds(base + off, CHUNK)])


def _tc_pad_body(u_ref, v_ref, up_ref, vp_ref):
    zpad = jnp.zeros((PAD_BLOCK, PAD_DIM - EMBED_DIM), jnp.float32)
    up_ref[...] = jnp.concatenate([u_ref[...], zpad], axis=1)
    vp_ref[...] = jnp.concatenate([v_ref[...], zpad], axis=1)


PAD_BLOCK = 2000


def _tc_dot_body(ue_ref, ve_ref, out_ref):
    prod = ue_ref[...] * ve_ref[...]
    dots = jnp.sum(prod, axis=-1)
    out_ref[...] = 1.0 / (1.0 + jnp.exp(-dots))


TC_BLOCK = 2048


@jax.jit
def _run(center, context, u, v):
    u_pad, v_pad = pl.pallas_call(
        _tc_pad_body,
        out_shape=(
            jax.ShapeDtypeStruct((VOCAB_SIZE, PAD_DIM), jnp.float32),
            jax.ShapeDtypeStruct((VOCAB_SIZE, PAD_DIM), jnp.float32),
        ),
        grid=(VOCAB_SIZE // PAD_BLOCK,),
        in_specs=[
            pl.BlockSpec((PAD_BLOCK, EMBED_DIM), lambda i: (i, 0)),
            pl.BlockSpec((PAD_BLOCK, EMBED_DIM), lambda i: (i, 0)),
        ],
        out_specs=(
            pl.BlockSpec((PAD_BLOCK, PAD_DIM), lambda i: (i, 0)),
            pl.BlockSpec((PAD_BLOCK, PAD_DIM), lambda i: (i, 0)),
        ),
    )(u, v)

    mesh = plsc.VectorSubcoreMesh(core_axis_name="c", subcore_axis_name="s")
    ue, ve = pl.kernel(
        _sc_gather_body,
        out_type=(
            jax.ShapeDtypeStruct((BATCH, PAD_DIM), jnp.float32),
            jax.ShapeDtypeStruct((BATCH, PAD_DIM), jnp.float32),
        ),
        mesh=mesh,
        scratch_types=[
            pltpu.VMEM((B_PER_W,), jnp.int32),
            pltpu.VMEM((B_PER_W,), jnp.int32),
            pltpu.VMEM((CHUNK, PAD_DIM), jnp.float32),
            pltpu.VMEM((CHUNK, PAD_DIM), jnp.float32),
            pltpu.SemaphoreType.DMA,
            pltpu.SemaphoreType.DMA,
        ],
    )(center, context, u_pad, v_pad)

    out = pl.pallas_call(
        _tc_dot_body,
        out_shape=jax.ShapeDtypeStruct((BATCH,), jnp.float32),
        grid=(BATCH // TC_BLOCK,),
        in_specs=[
            pl.BlockSpec((TC_BLOCK, PAD_DIM), lambda i: (i, 0)),
            pl.BlockSpec((TC_BLOCK, PAD_DIM), lambda i: (i, 0)),
        ],
        out_specs=pl.BlockSpec((TC_BLOCK,), lambda i: (i,)),
    )(ue, ve)
    return out


def kernel(center, context, U, V):
    return _run(center, context, U, V)
